# baseline (speedup 1.0000x reference)
import functools

import jax
import jax.numpy as jnp
from jax import lax
from jax.experimental import pallas as pl
from jax.experimental.pallas import tpu as pltpu
from jax.experimental.pallas import tpu_sc as plsc

B = 16384
L = 200
EMB = 64
NCLS = 100

NW = 32
SPW = B // NW
C = 4
NCHUNK = SPW // C
ROWS = C * L
SEGS = ((0, 128), (128, 72))
RUNROLL = 8
INV_L = 1.0 / L


V = 1000000
TBW = 400
TBLK = V // TBW
TPW = TBLK // NW
TREST = TBLK - TPW * NW


def _sc_transpose(tabT):
    mesh = plsc.VectorSubcoreMesh(core_axis_name="c", subcore_axis_name="s")

    @functools.partial(
        pl.kernel,
        out_type=jax.ShapeDtypeStruct((V, EMB), jnp.float32),
        mesh=mesh,
        compiler_params=pltpu.CompilerParams(
            use_tc_tiling_on_sc=False, needs_layout_passes=False),
        scratch_types=[
            pltpu.VMEM((EMB, TBW), jnp.float32),
            pltpu.VMEM((EMB, TBW), jnp.float32),
            pltpu.VMEM((TBW, EMB), jnp.float32),
            pltpu.VMEM((TBW, EMB), jnp.float32),
            pltpu.SemaphoreType.DMA,
            pltpu.SemaphoreType.DMA,
        ],
    )
    def tr_kernel(t_hbm, out_hbm, in0, in1, out0, out1, sem_i, sem_o):
        wid = lax.axis_index("s") * 2 + lax.axis_index("c")
        blk0 = wid * TPW
        iota16 = lax.iota(jnp.int32, 16)

        def in_copy(blk, dst):
            return pltpu.make_async_copy(
                t_hbm.at[:, pl.ds(blk * TBW, TBW)], dst, sem_i)

        def out_copy(blk, src):
            return pltpu.make_async_copy(
                src, out_hbm.at[pl.ds(blk * TBW, TBW)], sem_o)

        def transpose(inb, outb):
            def tbody(r, carry):
                rv = jnp.broadcast_to(r, (16,)).astype(jnp.int32)
                for k4 in range(4):
                    v = plsc.load_gather(inb, [iota16 + 16 * k4, rv])
                    outb[r, pl.ds(16 * k4, 16)] = v
                return carry
            lax.fori_loop(0, TBW, tbody, 0)

        in_copy(blk0, in0).start()
        in_copy(blk0 + 1, in1).start()

        def body(i, carry):
            b0 = blk0 + 2 * i
            in_copy(b0, in0).wait()

            @pl.when(i > 0)
            def _():
                out_copy(b0 - 2, out0).wait()

            transpose(in0, out0)
            out_copy(b0, out0).start()

            @pl.when(i < TPW // 2 - 1)
            def _():
                in_copy(b0 + 2, in0).start()

            in_copy(b0 + 1, in1).wait()

            @pl.when(i > 0)
            def _():
                out_copy(b0 - 1, out1).wait()

            transpose(in1, out1)
            out_copy(b0 + 1, out1).start()

            @pl.when(i < TPW // 2 - 1)
            def _():
                in_copy(b0 + 3, in1).start()

            return carry

        lax.fori_loop(0, TPW // 2, body, 0)
        out_copy(blk0 + TPW - 2, out0).wait()
        out_copy(blk0 + TPW - 1, out1).wait()

        @pl.when(wid < TREST)
        def _():
            blk = NW * TPW + wid
            in_copy(blk, in0).start()
            in_copy(blk, in0).wait()
            transpose(in0, out0)
            out_copy(blk, out0).start()
            out_copy(blk, out0).wait()

    return tr_kernel(tabT)


def _sc_pool(x, table):
    mesh = plsc.VectorSubcoreMesh(core_axis_name="c", subcore_axis_name="s")

    @functools.partial(
        pl.kernel,
        out_type=jax.ShapeDtypeStruct((B, 2 * EMB), jnp.float32),
        mesh=mesh,
        compiler_params=pltpu.CompilerParams(use_tc_tiling_on_sc=False),
        scratch_types=[
            pltpu.VMEM((C, L), jnp.int32),
            pltpu.VMEM((C, L), jnp.int32),
            pltpu.VMEM((ROWS, EMB), jnp.float32),
            pltpu.VMEM((ROWS, EMB), jnp.float32),
            pltpu.VMEM((C, 2 * EMB), jnp.float32),
            pltpu.VMEM((C, 2 * EMB), jnp.float32),
            pltpu.SemaphoreType.DMA,
            pltpu.SemaphoreType.DMA,
        ],
    )
    def sc_kernel(x_hbm, tab_hbm, out_hbm,
                  idx0, idx1, rows0, rows1, out0, out1, sem_i, sem_g):
        wid = lax.axis_index("s") * 2 + lax.axis_index("c")
        row0 = wid * SPW

        def idx_copy(chunk, dst):
            return pltpu.make_async_copy(
                x_hbm.at[pl.ds(row0 + chunk * C, C)], dst, sem_i)

        def fire(idxb, rowsb):
            for s in range(C):
                for off, n in SEGS:
                    pltpu.make_async_copy(
                        tab_hbm.at[idxb.at[s, pl.ds(off, n)]],
                        rowsb.at[pl.ds(s * L + off, n)],
                        sem_g).start()

        def wait_rows(idxb, rowsb):
            for s in range(C):
                for off, n in SEGS:
                    pltpu.make_async_copy(
                        tab_hbm.at[idxb.at[s, pl.ds(off, n)]],
                        rowsb.at[pl.ds(s * L + off, n)],
                        sem_g).wait()

        def reduce_store(rowsb, outb, chunk):
            for s in range(C):
                rbase = s * L

                def body(it, acc):
                    s0, s1, s2, s3, m0, m1, m2, m3 = acc
                    base = rbase + it * RUNROLL
                    for u in range(RUNROLL):
                        v0 = rowsb[base + u, pl.ds(0, 16)]
                        v1 = rowsb[base + u, pl.ds(16, 16)]
                        v2 = rowsb[base + u, pl.ds(32, 16)]
                        v3 = rowsb[base + u, pl.ds(48, 16)]
                        s0 = s0 + v0
                        s1 = s1 + v1
                        s2 = s2 + v2
                        s3 = s3 + v3
                        m0 = jnp.maximum(m0, v0)
                        m1 = jnp.maximum(m1, v1)
                        m2 = jnp.maximum(m2, v2)
                        m3 = jnp.maximum(m3, v3)
                    return (s0, s1, s2, s3, m0, m1, m2, m3)

                z = jnp.zeros((16,), jnp.float32)
                ninf = jnp.full((16,), -jnp.inf, jnp.float32)
                s0, s1, s2, s3, m0, m1, m2, m3 = lax.fori_loop(
                    0, L // RUNROLL, body, (z, z, z, z, ninf, ninf, ninf, ninf))
                outb[s, pl.ds(0, 16)] = s0 * INV_L
                outb[s, pl.ds(16, 16)] = s1 * INV_L
                outb[s, pl.ds(32, 16)] = s2 * INV_L
                outb[s, pl.ds(48, 16)] = s3 * INV_L
                outb[s, pl.ds(64, 16)] = m0
                outb[s, pl.ds(80, 16)] = m1
                outb[s, pl.ds(96, 16)] = m2
                outb[s, pl.ds(112, 16)] = m3
            pltpu.sync_copy(outb, out_hbm.at[pl.ds(row0 + chunk * C, C)])

        idx_copy(0, idx0).start()
        idx_copy(0, idx0).wait()
        idx_copy(1, idx1).start()
        fire(idx0, rows0)

        def body(i, carry):
            g0 = 2 * i
            wait_rows(idx0, rows0)
            idx_copy(g0 + 1, idx1).wait()

            @pl.when(i < NCHUNK // 2 - 1)
            def _():
                idx_copy(g0 + 2, idx0).start()

            fire(idx1, rows1)
            reduce_store(rows0, out0, g0)
            wait_rows(idx1, rows1)

            @pl.when(i < NCHUNK // 2 - 1)
            def _():
                idx_copy(g0 + 2, idx0).wait()
                idx_copy(g0 + 3, idx1).start()
                fire(idx0, rows0)

            reduce_store(rows1, out1, g0 + 1)
            return carry

        lax.fori_loop(0, NCHUNK // 2, body, 0)

    return sc_kernel(x, table)


BLK = 2048


def _mlp_body(p_ref, w1_ref, b1_ref, w2_ref, b2_ref, o_ref):
    h = jnp.dot(p_ref[...], w1_ref[...], preferred_element_type=jnp.float32)
    h = jnp.maximum(h + b1_ref[...], 0.0)
    o = jnp.dot(h, w2_ref[...], preferred_element_type=jnp.float32)
    o = o + b2_ref[...]
    m = jnp.max(o, axis=1, keepdims=True)
    ex = jnp.exp(o - m)
    o_ref[...] = o - m - jnp.log(jnp.sum(ex, axis=1, keepdims=True))


def _mlp(pooled, w1p, b1p, w2p, b2p):
    return pl.pallas_call(
        _mlp_body,
        grid=(B // BLK,),
        in_specs=[
            pl.BlockSpec((BLK, 2 * EMB), lambda i: (i, 0)),
            pl.BlockSpec((2 * EMB, 128), lambda i: (0, 0)),
            pl.BlockSpec((1, 128), lambda i: (0, 0)),
            pl.BlockSpec((128, 128), lambda i: (0, 0)),
            pl.BlockSpec((1, 128), lambda i: (0, 0)),
        ],
        out_specs=pl.BlockSpec((BLK, 128), lambda i: (i, 0)),
        out_shape=jax.ShapeDtypeStruct((B, 128), jnp.float32),
    )(pooled, w1p, b1p, w2p, b2p)


def kernel(x, table, W1, b1, W2, b2):
    table_rm = _sc_transpose(table.T)
    pooled = _sc_pool(x, table_rm)
    w1p = jnp.zeros((2 * EMB, 128), jnp.float32).at[:, :NCLS].set(W1)
    b1p = jnp.zeros((1, 128), jnp.float32).at[0, :NCLS].set(b1)
    w2p = jnp.zeros((128, 128), jnp.float32).at[:NCLS, :NCLS].set(W2)
    b2p = jnp.full((1, 128), -1e30, jnp.float32).at[0, :NCLS].set(b2)
    out = _mlp(pooled, w1p, b1p, w2p, b2p)
    return out[:, :NCLS]


# speedup vs baseline: 6.8332x; 6.8332x over previous
import functools

import jax
import jax.numpy as jnp
from jax import lax
from jax.experimental import pallas as pl
from jax.experimental.pallas import tpu as pltpu
from jax.experimental.pallas import tpu_sc as plsc

B = 16384
L = 200
EMB = 64
NCLS = 100

NW = 32
SPW = B // NW
C = 2
NCHUNK = SPW // C
NBUF = 4
ROWS = C * L
SEGS = ((0, 128), (128, 72))
RUNROLL = 8
INV_L = 1.0 / L


def _sc_pool(x, table):
    mesh = plsc.VectorSubcoreMesh(core_axis_name="c", subcore_axis_name="s")

    @functools.partial(
        pl.kernel,
        out_type=jax.ShapeDtypeStruct((B, 2 * EMB), jnp.float32),
        mesh=mesh,
        compiler_params=pltpu.CompilerParams(use_tc_tiling_on_sc=False),
        scratch_types=[
            pltpu.VMEM((NBUF, C, L), jnp.int32),
            pltpu.VMEM((NBUF, ROWS, EMB), jnp.float32),
            pltpu.VMEM((NBUF, C, 2 * EMB), jnp.float32),
            pltpu.SemaphoreType.DMA,
            pltpu.SemaphoreType.DMA,
            pltpu.SemaphoreType.DMA,
        ],
    )
    def sc_kernel(x_hbm, tab_hbm, out_hbm,
                  idxr, rowsr, outr, sem_i, sem_g, sem_o):
        wid = lax.axis_index("s") * 2 + lax.axis_index("c")
        row0 = wid * SPW

        def idx_copy(chunk, b):
            return pltpu.make_async_copy(
                x_hbm.at[pl.ds(row0 + chunk * C, C)], idxr.at[b], sem_i)

        def fire(b):
            for s in range(C):
                for off, n in SEGS:
                    pltpu.make_async_copy(
                        tab_hbm.at[idxr.at[b, s, pl.ds(off, n)]],
                        rowsr.at[b, pl.ds(s * L + off, n)],
                        sem_g).start()

        def wait_rows(b):
            for s in range(C):
                for off, n in SEGS:
                    pltpu.make_async_copy(
                        tab_hbm.at[idxr.at[b, s, pl.ds(off, n)]],
                        rowsr.at[b, pl.ds(s * L + off, n)],
                        sem_g).wait()

        def out_copy(chunk, b):
            return pltpu.make_async_copy(
                outr.at[b], out_hbm.at[pl.ds(row0 + chunk * C, C)], sem_o)

        def reduce_store(b, chunk):
            rowsb = rowsr.at[b]
            outb = outr.at[b]
            for s in range(C):
                rbase = s * L

                def body(it, acc):
                    s0, s1, s2, s3, m0, m1, m2, m3 = acc
                    base = rbase + it * RUNROLL
                    for u in range(RUNROLL):
                        v0 = rowsb[base + u, pl.ds(0, 16)]
                        v1 = rowsb[base + u, pl.ds(16, 16)]
                        v2 = rowsb[base + u, pl.ds(32, 16)]
                        v3 = rowsb[base + u, pl.ds(48, 16)]
                        s0 = s0 + v0
                        s1 = s1 + v1
                        s2 = s2 + v2
                        s3 = s3 + v3
                        m0 = jnp.maximum(m0, v0)
                        m1 = jnp.maximum(m1, v1)
                        m2 = jnp.maximum(m2, v2)
                        m3 = jnp.maximum(m3, v3)
                    return (s0, s1, s2, s3, m0, m1, m2, m3)

                z = jnp.zeros((16,), jnp.float32)
                ninf = jnp.full((16,), -jnp.inf, jnp.float32)
                s0, s1, s2, s3, m0, m1, m2, m3 = lax.fori_loop(
                    0, L // RUNROLL, body, (z, z, z, z, ninf, ninf, ninf, ninf))
                outb[s, pl.ds(0, 16)] = s0 * INV_L
                outb[s, pl.ds(16, 16)] = s1 * INV_L
                outb[s, pl.ds(32, 16)] = s2 * INV_L
                outb[s, pl.ds(48, 16)] = s3 * INV_L
                outb[s, pl.ds(64, 16)] = m0
                outb[s, pl.ds(80, 16)] = m1
                outb[s, pl.ds(96, 16)] = m2
                outb[s, pl.ds(112, 16)] = m3
            out_copy(chunk, b).start()

        idx_copy(0, 0).start()
        idx_copy(0, 0).wait()
        idx_copy(1, 1).start()
        idx_copy(1, 1).wait()
        fire(0)
        fire(1)
        idx_copy(2, 2).start()
        idx_copy(3, 3).start()

        def body(i, carry):
            for j in range(NBUF):
                g = NBUF * i + j
                wait_rows(j)

                @pl.when(g + 2 < NCHUNK)
                def _():
                    idx_copy(g + 2, (j + 2) % NBUF).wait()
                    fire((j + 2) % NBUF)

                @pl.when(g + 4 < NCHUNK)
                def _():
                    idx_copy(g + 4, j).start()

                @pl.when(g >= NBUF)
                def _():
                    out_copy(g - NBUF, j).wait()

                reduce_store(j, g)
            return carry

        lax.fori_loop(0, NCHUNK // NBUF, body, 0)
        for j in range(NBUF):
            out_copy(NCHUNK - NBUF + j, j).wait()

    return sc_kernel(x, table)


BLK = 2048


def _mlp_body(p_ref, w1_ref, b1_ref, w2_ref, b2_ref, o_ref):
    h = jnp.dot(p_ref[...], w1_ref[...], preferred_element_type=jnp.float32)
    h = jnp.maximum(h + b1_ref[...], 0.0)
    o = jnp.dot(h, w2_ref[...], preferred_element_type=jnp.float32)
    o = o + b2_ref[...]
    m = jnp.max(o, axis=1, keepdims=True)
    ex = jnp.exp(o - m)
    o_ref[...] = o - m - jnp.log(jnp.sum(ex, axis=1, keepdims=True))


def _mlp(pooled, w1p, b1p, w2p, b2p):
    return pl.pallas_call(
        _mlp_body,
        grid=(B // BLK,),
        in_specs=[
            pl.BlockSpec((BLK, 2 * EMB), lambda i: (i, 0)),
            pl.BlockSpec((2 * EMB, 128), lambda i: (0, 0)),
            pl.BlockSpec((1, 128), lambda i: (0, 0)),
            pl.BlockSpec((128, 128), lambda i: (0, 0)),
            pl.BlockSpec((1, 128), lambda i: (0, 0)),
        ],
        out_specs=pl.BlockSpec((BLK, 128), lambda i: (i, 0)),
        out_shape=jax.ShapeDtypeStruct((B, 128), jnp.float32),
    )(pooled, w1p, b1p, w2p, b2p)


def kernel(x, table, W1, b1, W2, b2):
    pooled = _sc_pool(x, table)
    w1p = jnp.zeros((2 * EMB, 128), jnp.float32).at[:, :NCLS].set(W1)
    b1p = jnp.zeros((1, 128), jnp.float32).at[0, :NCLS].set(b1)
    w2p = jnp.zeros((128, 128), jnp.float32).at[:NCLS, :NCLS].set(W2)
    b2p = jnp.full((1, 128), -1e30, jnp.float32).at[0, :NCLS].set(b2)
    out = _mlp(pooled, w1p, b1p, w2p, b2p)
    return out[:, :NCLS]
